# baseline (speedup 1.0000x reference)
import functools

import jax
import jax.numpy as jnp
from jax import lax
from jax.experimental import pallas as pl
from jax.experimental.pallas import tpu as pltpu
from jax.experimental.pallas import tpu_sc as plsc

B = 16
M = 10000
FIN = 128
NK = 6
FOUT = 64
E = 320000
M_PAD = 10240

NCORE = 2
NTILE = 16
NP = 8
FW = 2 * FOUT
SUB = 128
SPT = 160
E_PAD = NTILE * SPT * SUB
HSPT = SPT // 8
NSUB_B = 1
BLK = NSUB_B * SUB
NBLK_H = HSPT // NSUB_B
NPAIR = NBLK_H // 2
RPT = M_PAD // NTILE
DR = 16
NDR = RPT // DR
CPC = NP // NCORE
NLANE = 16


def _spmm_body(has_sub, scale2,
               x_ref, cols_ref, rows_ref, vals_ref, c_ref, sub_ref, out_ref,
               acc, gbuf0, gbuf1, colsv, rowsv, valv, stage, subv,
               gsem0, gsem1):
    ci = lax.axis_index("c")
    si = lax.axis_index("s")

    NH = 8

    def chunk_body(c, _):
        cc = ci * CPC + c
        r0 = si * RPT
        pltpu.sync_copy(c_ref.at[cc, pl.ds(r0, RPT)], acc.at[pl.ds(r0, RPT)])
        plsc.subcore_barrier()

        xc = x_ref.at[cc]

        def h_body(h, _):
            sub0 = si * SPT + h * HSPT
            pltpu.sync_copy(cols_ref.at[pl.ds(sub0, HSPT)], colsv)
            pltpu.sync_copy(rows_ref.at[pl.ds(sub0, HSPT)], rowsv)
            pltpu.sync_copy(vals_ref.at[pl.ds(sub0 * SUB, HSPT * SUB)], valv)

            def fire(b, buf, sem):
                for j in range(NSUB_B):
                    pltpu.async_copy(xc.at[colsv.at[b * NSUB_B + j]],
                                     buf.at[pl.ds(j * SUB, SUB)], sem)

            def process(b, buf, sem):
                pltpu.make_async_copy(xc.at[pl.ds(0, BLK)], buf, sem).wait()

                @plsc.parallel_loop(0, BLK // NLANE, unroll=2)
                def mul_body(g):
                    vv = valv[pl.ds(b * BLK + g * NLANE, NLANE)]
                    vs = vv * 2.0 if scale2 else vv
                    for e in range(NLANE):
                        sv = jnp.take(vs, jnp.full((NLANE,), e, jnp.int32))
                        i = g * NLANE + e
                        for v in range(FW // NLANE):
                            buf[i, pl.ds(v * NLANE, NLANE)] = (
                                buf[i, pl.ds(v * NLANE, NLANE)] * sv)

                for j in range(NSUB_B):
                    pltpu.sync_copy(buf.at[pl.ds(j * SUB, SUB)],
                                    acc.at[rowsv.at[b * NSUB_B + j]], add=True)

            fire(0, gbuf0, gsem0)

            def pair_body(g, _):
                b0 = 2 * g
                fire(b0 + 1, gbuf1, gsem1)
                process(b0, gbuf0, gsem0)

                @pl.when(g < NPAIR - 1)
                def _():
                    fire(b0 + 2, gbuf0, gsem0)
                process(b0 + 1, gbuf1, gsem1)
                return 0
            lax.fori_loop(0, NPAIR, pair_body, 0)
            return 0
        lax.fori_loop(0, NH, h_body, 0)
        plsc.subcore_barrier()

        for p in range(NDR):
            rp = r0 + p * DR
            pltpu.sync_copy(acc.at[pl.ds(rp, DR)], stage)
            if has_sub:
                pltpu.sync_copy(sub_ref.at[cc, pl.ds(rp, DR)], subv)

                def sub_body(i, _):
                    for v in range(FW // NLANE):
                        stage[i, pl.ds(v * NLANE, NLANE)] = (
                            stage[i, pl.ds(v * NLANE, NLANE)]
                            - subv[i, pl.ds(v * NLANE, NLANE)])
                    return 0
                lax.fori_loop(0, DR, sub_body, 0)
            pltpu.sync_copy(stage, out_ref.at[cc, pl.ds(rp, DR)])
        plsc.subcore_barrier()
        return 0

    lax.fori_loop(0, CPC, chunk_body, 0)


@functools.lru_cache(maxsize=None)
def _make_spmm(has_sub, scale2):
    mesh = plsc.VectorSubcoreMesh(core_axis_name="c", subcore_axis_name="s")
    body = functools.partial(_spmm_body, has_sub, scale2)
    k = pl.kernel(
        body,
        out_type=jax.ShapeDtypeStruct((NP, M_PAD, FW), jnp.float32),
        mesh=mesh,
        scratch_types=[
            pltpu.VMEM_SHARED((M_PAD, FW), jnp.float32),
            pltpu.VMEM((BLK, FW), jnp.float32),
            pltpu.VMEM((BLK, FW), jnp.float32),
            pltpu.VMEM((HSPT, SUB), jnp.int32),
            pltpu.VMEM((HSPT, SUB), jnp.int32),
            pltpu.VMEM((HSPT * SUB,), jnp.float32),
            pltpu.VMEM((DR, FW), jnp.float32),
            pltpu.VMEM((DR, FW), jnp.float32),
            pltpu.SemaphoreType.DMA,
            pltpu.SemaphoreType.DMA,
        ],
        compiler_params=pltpu.CompilerParams(use_tc_tiling_on_sc=False),
        name=f"cheb_spmm_sub{int(has_sub)}_s{int(scale2)}",
    )

    def call(X, cols2d, rows2d, vals_p, C, Sub):
        return k(X, cols2d, rows2d, vals_p, C, Sub)
    return call


def _proj_body(x_ref, wt_ref, out_ref):
    for kk in range(NK):
        out_ref[kk, 0] = jnp.concatenate(
            [jnp.dot(x_ref[0], wt_ref[kk],
                     preferred_element_type=jnp.float32),
             jnp.dot(x_ref[1], wt_ref[kk],
                     preferred_element_type=jnp.float32)], axis=1)


_MB = 2048


def _project(x, wt):
    return pl.pallas_call(
        _proj_body,
        grid=(NP, M_PAD // _MB),
        in_specs=[
            pl.BlockSpec((2, _MB, FIN), lambda p, i: (p, i, 0)),
            pl.BlockSpec((NK, FIN, FOUT), lambda p, i: (0, 0, 0)),
        ],
        out_specs=pl.BlockSpec((NK, 1, _MB, FW), lambda p, i: (0, p, i, 0)),
        out_shape=jax.ShapeDtypeStruct((NK, NP, M_PAD, FW), jnp.float32),
    )(x, wt)


def kernel(x, L_rows, L_cols, L_vals, W):
    wt = jnp.transpose(W.reshape(FIN, NK, FOUT), (1, 0, 2))
    x_p = jnp.pad(x, ((0, 0), (0, M_PAD - M), (0, 0)))
    C = _project(x_p, wt)

    pad = E_PAD - E
    cols2d = jnp.pad(L_cols, (0, pad)).reshape(E_PAD // SUB, SUB)
    rows2d = jnp.pad(L_rows, (0, pad)).reshape(E_PAD // SUB, SUB)
    vals_p = jnp.pad(L_vals, (0, pad))

    spmm_first = _make_spmm(False, True)
    spmm_mid = _make_spmm(True, True)
    spmm_last = _make_spmm(True, False)

    b5 = C[5]
    b4 = spmm_first(b5, cols2d, rows2d, vals_p, C[4], b5)
    b3 = spmm_mid(b4, cols2d, rows2d, vals_p, C[3], b5)
    b2 = spmm_mid(b3, cols2d, rows2d, vals_p, C[2], b4)
    b1 = spmm_mid(b2, cols2d, rows2d, vals_p, C[1], b3)
    S = spmm_last(b1, cols2d, rows2d, vals_p, C[0], b2)
    S = S.reshape(NP, M_PAD, 2, FOUT).transpose(0, 2, 1, 3)
    return S.reshape(B, M_PAD, FOUT)[:, :M, :]


# speedup vs baseline: 1.4880x; 1.4880x over previous
import functools

import jax
import jax.numpy as jnp
from jax import lax
from jax.experimental import pallas as pl
from jax.experimental.pallas import tpu as pltpu
from jax.experimental.pallas import tpu_sc as plsc

B = 16
M = 10000
FIN = 128
NK = 6
FOUT = 64
E = 320000
M_PAD = 10240

NCORE = 2
NTILE = 16
SUB = 128
SPT = 160
E_PAD = NTILE * SPT * SUB
HSPT = SPT // 4
NSUB_B = 1
BLK = NSUB_B * SUB
NBLK_H = HSPT // NSUB_B
NPAIR = NBLK_H // 2
RPT = M_PAD // NTILE
DR = 64
NDR = RPT // DR
CPC = B // NCORE
NLANE = 16


def _spmm_body(has_sub, scale2,
               x_ref, cols_ref, rows_ref, vals_ref, c_ref, sub_ref, out_ref,
               acc, xcache, gbuf0, gbuf1, colsv, rowsv, valv, stage, subv,
               gsem0, gsem1):
    ci = lax.axis_index("c")
    si = lax.axis_index("s")

    def chunk_body(c, _):
        cc = ci * CPC + c
        r0 = si * RPT
        pltpu.sync_copy(c_ref.at[cc, pl.ds(r0, RPT)], acc.at[pl.ds(r0, RPT)])
        pltpu.sync_copy(x_ref.at[cc, pl.ds(r0, RPT)],
                        xcache.at[pl.ds(r0, RPT)])
        plsc.subcore_barrier()

        def h_body(h, _):
            sub0 = si * SPT + h * HSPT
            pltpu.sync_copy(cols_ref.at[pl.ds(sub0, HSPT)], colsv)
            pltpu.sync_copy(rows_ref.at[pl.ds(sub0, HSPT)], rowsv)
            pltpu.sync_copy(vals_ref.at[pl.ds(sub0 * SUB, HSPT * SUB)], valv)

            def fire(b, buf, sem):
                for j in range(NSUB_B):
                    pltpu.async_copy(xcache.at[colsv.at[b * NSUB_B + j]],
                                     buf.at[pl.ds(j * SUB, SUB)], sem)

            def process(b, buf, sem):
                pltpu.make_async_copy(xcache.at[pl.ds(0, BLK)], buf,
                                      sem).wait()

                @plsc.parallel_loop(0, BLK // NLANE, unroll=4)
                def mul_body(g):
                    vv = valv[pl.ds(b * BLK + g * NLANE, NLANE)]
                    vs = vv * 2.0 if scale2 else vv
                    for e in range(NLANE):
                        sv = jnp.take(vs, jnp.full((NLANE,), e, jnp.int32))
                        i = g * NLANE + e
                        for v in range(FOUT // NLANE):
                            buf[i, pl.ds(v * NLANE, NLANE)] = (
                                buf[i, pl.ds(v * NLANE, NLANE)] * sv)

                for j in range(NSUB_B):
                    pltpu.sync_copy(buf.at[pl.ds(j * SUB, SUB)],
                                    acc.at[rowsv.at[b * NSUB_B + j]], add=True)

            fire(0, gbuf0, gsem0)

            def pair_body(g, _):
                b0 = 2 * g
                fire(b0 + 1, gbuf1, gsem1)
                process(b0, gbuf0, gsem0)

                @pl.when(g < NPAIR - 1)
                def _():
                    fire(b0 + 2, gbuf0, gsem0)
                process(b0 + 1, gbuf1, gsem1)
                return 0
            lax.fori_loop(0, NPAIR, pair_body, 0)
            return 0
        lax.fori_loop(0, 4, h_body, 0)
        plsc.subcore_barrier()

        for p in range(NDR):
            rp = r0 + p * DR
            pltpu.sync_copy(acc.at[pl.ds(rp, DR)], stage)
            if has_sub:
                pltpu.sync_copy(sub_ref.at[cc, pl.ds(rp, DR)], subv)

                def sub_body(i, _):
                    for v in range(FOUT // NLANE):
                        stage[i, pl.ds(v * NLANE, NLANE)] = (
                            stage[i, pl.ds(v * NLANE, NLANE)]
                            - subv[i, pl.ds(v * NLANE, NLANE)])
                    return 0
                lax.fori_loop(0, DR, sub_body, 0)
            pltpu.sync_copy(stage, out_ref.at[cc, pl.ds(rp, DR)])
        plsc.subcore_barrier()
        return 0

    lax.fori_loop(0, CPC, chunk_body, 0)


@functools.lru_cache(maxsize=None)
def _make_spmm(has_sub, scale2):
    mesh = plsc.VectorSubcoreMesh(core_axis_name="c", subcore_axis_name="s")
    body = functools.partial(_spmm_body, has_sub, scale2)
    k = pl.kernel(
        body,
        out_type=jax.ShapeDtypeStruct((B, M_PAD, FOUT), jnp.float32),
        mesh=mesh,
        scratch_types=[
            pltpu.VMEM_SHARED((M_PAD, FOUT), jnp.float32),
            pltpu.VMEM_SHARED((M_PAD, FOUT), jnp.float32),
            pltpu.VMEM((BLK, FOUT), jnp.float32),
            pltpu.VMEM((BLK, FOUT), jnp.float32),
            pltpu.VMEM((HSPT, SUB), jnp.int32),
            pltpu.VMEM((HSPT, SUB), jnp.int32),
            pltpu.VMEM((HSPT * SUB,), jnp.float32),
            pltpu.VMEM((DR, FOUT), jnp.float32),
            pltpu.VMEM((DR, FOUT), jnp.float32),
            pltpu.SemaphoreType.DMA,
            pltpu.SemaphoreType.DMA,
        ],
        compiler_params=pltpu.CompilerParams(use_tc_tiling_on_sc=False),
        name=f"cheb_spmm_sub{int(has_sub)}_s{int(scale2)}",
    )

    def call(X, cols2d, rows2d, vals_p, C, Sub):
        return k(X, cols2d, rows2d, vals_p, C, Sub)
    return call


def _proj_body(x_ref, wt_ref, out_ref):
    xb = x_ref[0]
    for kk in range(NK):
        out_ref[kk, 0] = jnp.dot(xb, wt_ref[kk],
                                 preferred_element_type=jnp.float32)


_MB = 2048


def _project(x, wt):
    return pl.pallas_call(
        _proj_body,
        grid=(B, M_PAD // _MB),
        in_specs=[
            pl.BlockSpec((1, _MB, FIN), lambda b, i: (b, i, 0)),
            pl.BlockSpec((NK, FIN, FOUT), lambda b, i: (0, 0, 0)),
        ],
        out_specs=pl.BlockSpec((NK, 1, _MB, FOUT), lambda b, i: (0, b, i, 0)),
        out_shape=jax.ShapeDtypeStruct((NK, B, M_PAD, FOUT), jnp.float32),
    )(x, wt)


def kernel(x, L_rows, L_cols, L_vals, W):
    wt = jnp.transpose(W.reshape(FIN, NK, FOUT), (1, 0, 2))
    x_p = jnp.pad(x, ((0, 0), (0, M_PAD - M), (0, 0)))
    C = _project(x_p, wt)

    pad = E_PAD - E
    cols2d = jnp.pad(L_cols, (0, pad)).reshape(E_PAD // SUB, SUB)
    rows2d = jnp.pad(L_rows, (0, pad)).reshape(E_PAD // SUB, SUB)
    vals_p = jnp.pad(L_vals, (0, pad))

    spmm_first = _make_spmm(False, True)
    spmm_mid = _make_spmm(True, True)
    spmm_last = _make_spmm(True, False)

    b5 = C[5]
    b4 = spmm_first(b5, cols2d, rows2d, vals_p, C[4], b5)
    b3 = spmm_mid(b4, cols2d, rows2d, vals_p, C[3], b5)
    b2 = spmm_mid(b3, cols2d, rows2d, vals_p, C[2], b4)
    b1 = spmm_mid(b2, cols2d, rows2d, vals_p, C[1], b3)
    S = spmm_last(b1, cols2d, rows2d, vals_p, C[0], b2)
    return S[:, :M, :]
